# baseline (speedup 1.0000x reference)
import functools

import jax
import jax.numpy as jnp
from jax import lax
from jax.experimental import pallas as pl
from jax.experimental.pallas import tpu as pltpu
from jax.experimental.pallas import tpu_sc as plsc

_K_CAT = 8192
_DIM = 256
_BETA = 0.25
_N_TOK = 9216

_BN = 192

_BNC = 1024

_SC_NC = 2
_SC_NS = 16
_SC_NW = _SC_NC * _SC_NS
_B_PER_W = _N_TOK // _SC_NW
_CHUNK = 96
_NCH = _B_PER_W // _CHUNK


def _dist_dot(ze_ref, ej_ref):
    ze2 = ze_ref[...] + ze_ref[...]
    return lax.dot_general(
        ze2, ej_ref[...],
        (((1,), (1,)), ((), ())),
        preferred_element_type=jnp.float32,
    )


def _argmin_epilogue(a_ref, b_ref, colf_ref, dot2, ids_ref):
    d = (a_ref[...] + b_ref[...]) - dot2
    lmin = jnp.min(d, axis=1, keepdims=True)
    masked = jnp.where(d == lmin, colf_ref[...], jnp.float32(3e38))
    first = jnp.min(masked, axis=1, keepdims=True)
    ids_ref[...] = first.astype(jnp.int32)


def _argmin_body(a_ref, b_ref, colf_ref, ze_ref, ej_ref, ids_ref, sa, sb):
    n = pl.program_id(0)
    nn = pl.num_programs(0)
    even = (n % 2) == 0

    @pl.when(jnp.logical_and(n < nn - 1, even))
    def _dot_even():
        sa[...] = _dist_dot(ze_ref, ej_ref)

    @pl.when(jnp.logical_and(n < nn - 1, jnp.logical_not(even)))
    def _dot_odd():
        sb[...] = _dist_dot(ze_ref, ej_ref)

    @pl.when(jnp.logical_and(n > 0, jnp.logical_not(even)))
    def _ep_even():
        _argmin_epilogue(a_ref, b_ref, colf_ref, sa[...], ids_ref)

    @pl.when(jnp.logical_and(n > 0, even))
    def _ep_odd():
        _argmin_epilogue(a_ref, b_ref, colf_ref, sb[...], ids_ref)


def _compute_ids(a, bnorm, colf, ze, ej):
    num_n = _N_TOK // _BN
    return pl.pallas_call(
        _argmin_body,
        grid=(num_n + 1,),
        in_specs=[
            pl.BlockSpec((_BN, 1), lambda n: (jnp.maximum(n - 1, 0), 0)),
            pl.BlockSpec((1, _K_CAT), lambda n: (0, 0)),
            pl.BlockSpec((1, _K_CAT), lambda n: (0, 0)),
            pl.BlockSpec((_BN, _DIM), lambda n: (jnp.minimum(n, num_n - 1), 0)),
            pl.BlockSpec((_K_CAT, _DIM), lambda n: (0, 0)),
        ],
        out_specs=pl.BlockSpec((_BN, 1), lambda n: (jnp.maximum(n - 1, 0), 0)),
        out_shape=jax.ShapeDtypeStruct((_N_TOK, 1), jnp.int32),
        scratch_shapes=[
            pltpu.VMEM((_BN, _K_CAT), jnp.float32),
            pltpu.VMEM((_BN, _K_CAT), jnp.float32),
        ],
        compiler_params=pltpu.CompilerParams(
            dimension_semantics=("arbitrary",),
        ),
    )(a, bnorm, colf, ze, ej)


def _gather_rows(emb, ids):
    mesh = plsc.VectorSubcoreMesh(core_axis_name="c", subcore_axis_name="s")

    @functools.partial(
        pl.kernel,
        mesh=mesh,
        out_type=jax.ShapeDtypeStruct((_N_TOK, _DIM), jnp.float32),
        scratch_types=[
            pltpu.VMEM((_NCH, _CHUNK), jnp.int32),
            pltpu.VMEM((_NCH, _CHUNK, _DIM), jnp.float32),
            pltpu.SemaphoreType.DMA,
        ],
    )
    def _sc_gather(table_hbm, idx_hbm, out_hbm, idx_v, rows_v, sem):
        wid = lax.axis_index("s") * _SC_NC + lax.axis_index("c")
        base = wid * _B_PER_W
        for j in range(_NCH):
            pltpu.sync_copy(idx_hbm.at[pl.ds(base + j * _CHUNK, _CHUNK)], idx_v.at[j])
        copies = [
            pltpu.async_copy(table_hbm.at[idx_v.at[j]], rows_v.at[j], sem)
            for j in range(_NCH)
        ]
        for c in copies:
            c.wait()
        for j in range(_NCH):
            pltpu.sync_copy(rows_v.at[j], out_hbm.at[pl.ds(base + j * _CHUNK, _CHUNK)])

    return _sc_gather(emb, ids)


def _st_loss_body(ze_ref, zq_ref, out_ref, loss_ref, acc_ref):
    n = pl.program_id(0)
    ze = ze_ref[...]
    zq = zq_ref[...]
    out_ref[...] = (zq - ze) + ze
    part = jnp.sum((ze - zq) ** 2)
    acc_ref[0] = jnp.where(n == 0, part, acc_ref[0] + part)
    m = acc_ref[0] / jnp.float32(_N_TOK * _DIM)
    loss_ref[...] = jnp.broadcast_to(m + _BETA * m, (1, 1))


def _st_loss(ze, zq_rows):
    num_n = _N_TOK // _BNC
    return pl.pallas_call(
        _st_loss_body,
        grid=(num_n,),
        in_specs=[
            pl.BlockSpec((_BNC, _DIM), lambda n: (n, 0)),
            pl.BlockSpec((_BNC, _DIM), lambda n: (n, 0)),
        ],
        out_specs=[
            pl.BlockSpec((_BNC, _DIM), lambda n: (n, 0)),
            pl.BlockSpec((1, 1), lambda n: (0, 0)),
        ],
        out_shape=[
            jax.ShapeDtypeStruct((_N_TOK, _DIM), jnp.float32),
            jax.ShapeDtypeStruct((1, 1), jnp.float32),
        ],
        scratch_shapes=[pltpu.SMEM((1,), jnp.float32)],
    )(ze, zq_rows)


def kernel(x, emb_weight):
    b, c, h, w = x.shape
    ze = jnp.transpose(x, (0, 2, 3, 1)).reshape(-1, c)
    a = jnp.sum(ze ** 2, axis=-1, keepdims=True)
    bnorm = jnp.sum(emb_weight ** 2, axis=-1).reshape(1, -1)
    colf = jnp.arange(_K_CAT, dtype=jnp.float32).reshape(1, -1)
    ids = _compute_ids(a, bnorm, colf, ze, emb_weight).reshape(-1)
    zq_rows = _gather_rows(emb_weight, ids)
    zq_out_rows, loss = _st_loss(ze, zq_rows)
    zq_out = zq_out_rows.reshape(b, h, w, c).transpose(0, 3, 1, 2)
    return (zq_out, loss.reshape(()))


# speedup vs baseline: 1.2997x; 1.2997x over previous
import functools

import jax
import jax.numpy as jnp
from jax import lax
from jax.experimental import pallas as pl
from jax.experimental.pallas import tpu as pltpu
from jax.experimental.pallas import tpu_sc as plsc

_K_CAT = 8192
_DIM = 256
_BETA = 0.25
_N_TOK = 9216

_BN = 512

_BNC = 1024

_SC_NC = 2
_SC_NS = 16
_SC_NW = _SC_NC * _SC_NS
_B_PER_W = _N_TOK // _SC_NW
_CHUNK = 96
_NCH = _B_PER_W // _CHUNK


def _argmin_body(a_ref, b_ref, colf_ref, ze_ref, ej_ref, ids_ref):
    ze2 = ze_ref[...] + ze_ref[...]
    dot2 = lax.dot_general(
        ze2, ej_ref[...],
        (((1,), (1,)), ((), ())),
        preferred_element_type=jnp.float32,
    )
    d = (a_ref[...] + b_ref[...]) - dot2
    lmin = jnp.min(d, axis=1, keepdims=True)
    masked = jnp.where(d == lmin, colf_ref[...], jnp.float32(3e38))
    first = jnp.min(masked, axis=1, keepdims=True)
    ids_ref[...] = first.astype(jnp.int32)


def _compute_ids(a, bnorm, colf, ze, ej):
    num_n = _N_TOK // _BN
    return pl.pallas_call(
        _argmin_body,
        grid=(num_n,),
        in_specs=[
            pl.BlockSpec((_BN, 1), lambda n: (n, 0)),
            pl.BlockSpec((1, _K_CAT), lambda n: (0, 0)),
            pl.BlockSpec((1, _K_CAT), lambda n: (0, 0)),
            pl.BlockSpec((_BN, _DIM), lambda n: (n, 0)),
            pl.BlockSpec((_K_CAT, _DIM), lambda n: (0, 0)),
        ],
        out_specs=pl.BlockSpec((_BN, 1), lambda n: (n, 0)),
        out_shape=jax.ShapeDtypeStruct((_N_TOK, 1), jnp.int32),
        compiler_params=pltpu.CompilerParams(
            dimension_semantics=("parallel",),
        ),
    )(a, bnorm, colf, ze, ej)


def _gather_rows(emb, ids):
    mesh = plsc.VectorSubcoreMesh(core_axis_name="c", subcore_axis_name="s")

    @functools.partial(
        pl.kernel,
        mesh=mesh,
        out_type=jax.ShapeDtypeStruct((_N_TOK, _DIM), jnp.float32),
        scratch_types=[
            pltpu.VMEM((_NCH, _CHUNK), jnp.int32),
            pltpu.VMEM((_NCH, _CHUNK, _DIM), jnp.float32),
            pltpu.SemaphoreType.DMA,
        ],
    )
    def _sc_gather(table_hbm, idx_hbm, out_hbm, idx_v, rows_v, sem):
        wid = lax.axis_index("s") * _SC_NC + lax.axis_index("c")
        base = wid * _B_PER_W
        for j in range(_NCH):
            pltpu.sync_copy(idx_hbm.at[pl.ds(base + j * _CHUNK, _CHUNK)], idx_v.at[j])
        copies = [
            pltpu.async_copy(table_hbm.at[idx_v.at[j]], rows_v.at[j], sem)
            for j in range(_NCH)
        ]
        for c in copies:
            c.wait()
        for j in range(_NCH):
            pltpu.sync_copy(rows_v.at[j], out_hbm.at[pl.ds(base + j * _CHUNK, _CHUNK)])

    return _sc_gather(emb, ids)


def _st_loss_body(ze_ref, zq_ref, out_ref, loss_ref, acc_ref):
    n = pl.program_id(0)
    ze = ze_ref[...]
    zq = zq_ref[...]
    out_ref[...] = (zq - ze) + ze
    part = jnp.sum((ze - zq) ** 2)
    acc_ref[0] = jnp.where(n == 0, part, acc_ref[0] + part)
    m = acc_ref[0] / jnp.float32(_N_TOK * _DIM)
    loss_ref[...] = jnp.broadcast_to(m + _BETA * m, (1, 1))


def _st_loss(ze, zq_rows):
    num_n = _N_TOK // _BNC
    return pl.pallas_call(
        _st_loss_body,
        grid=(num_n,),
        in_specs=[
            pl.BlockSpec((_BNC, _DIM), lambda n: (n, 0)),
            pl.BlockSpec((_BNC, _DIM), lambda n: (n, 0)),
        ],
        out_specs=[
            pl.BlockSpec((_BNC, _DIM), lambda n: (n, 0)),
            pl.BlockSpec((1, 1), lambda n: (0, 0)),
        ],
        out_shape=[
            jax.ShapeDtypeStruct((_N_TOK, _DIM), jnp.float32),
            jax.ShapeDtypeStruct((1, 1), jnp.float32),
        ],
        scratch_shapes=[pltpu.SMEM((1,), jnp.float32)],
    )(ze, zq_rows)


def kernel(x, emb_weight):
    b, c, h, w = x.shape
    ze = jnp.transpose(x, (0, 2, 3, 1)).reshape(-1, c)
    a = jnp.sum(ze ** 2, axis=-1, keepdims=True)
    bnorm = jnp.sum(emb_weight ** 2, axis=-1).reshape(1, -1)
    colf = jnp.arange(_K_CAT, dtype=jnp.float32).reshape(1, -1)
    ids = _compute_ids(a, bnorm, colf, ze, emb_weight).reshape(-1)
    zq_rows = _gather_rows(emb_weight, ids)
    zq_out_rows, loss = _st_loss(ze, zq_rows)
    zq_out = zq_out_rows.reshape(b, h, w, c).transpose(0, 3, 1, 2)
    return (zq_out, loss.reshape(()))


# speedup vs baseline: 1.3355x; 1.0275x over previous
import functools

import jax
import jax.numpy as jnp
from jax import lax
from jax.experimental import pallas as pl
from jax.experimental.pallas import tpu as pltpu
from jax.experimental.pallas import tpu_sc as plsc

_K_CAT = 8192
_DIM = 256
_BETA = 0.25
_N_TOK = 9216

_BN = 576

_BNC = 2304

_SC_NC = 2
_SC_NS = 16
_SC_NW = _SC_NC * _SC_NS
_B_PER_W = _N_TOK // _SC_NW
_CHUNK = 96
_NCH = _B_PER_W // _CHUNK


def _argmin_body(a_ref, b_ref, colf_ref, ze_ref, ej_ref, ids_ref):
    ze2 = ze_ref[...] + ze_ref[...]
    dot2 = lax.dot_general(
        ze2, ej_ref[...],
        (((1,), (1,)), ((), ())),
        preferred_element_type=jnp.float32,
    )
    d = (a_ref[...] + b_ref[...]) - dot2
    lmin = jnp.min(d, axis=1, keepdims=True)
    masked = jnp.where(d == lmin, colf_ref[...], jnp.float32(3e38))
    first = jnp.min(masked, axis=1, keepdims=True)
    ids_ref[...] = first.astype(jnp.int32)


def _compute_ids(a, bnorm, colf, ze, ej):
    num_n = _N_TOK // _BN
    return pl.pallas_call(
        _argmin_body,
        grid=(num_n,),
        in_specs=[
            pl.BlockSpec((_BN, 1), lambda n: (n, 0)),
            pl.BlockSpec((1, _K_CAT), lambda n: (0, 0)),
            pl.BlockSpec((1, _K_CAT), lambda n: (0, 0)),
            pl.BlockSpec((_BN, _DIM), lambda n: (n, 0)),
            pl.BlockSpec((_K_CAT, _DIM), lambda n: (0, 0)),
        ],
        out_specs=pl.BlockSpec((_BN, 1), lambda n: (n, 0)),
        out_shape=jax.ShapeDtypeStruct((_N_TOK, 1), jnp.int32),
        compiler_params=pltpu.CompilerParams(
            dimension_semantics=("parallel",),
        ),
    )(a, bnorm, colf, ze, ej)


def _gather_rows(emb, ids):
    mesh = plsc.VectorSubcoreMesh(core_axis_name="c", subcore_axis_name="s")

    @functools.partial(
        pl.kernel,
        mesh=mesh,
        out_type=jax.ShapeDtypeStruct((_N_TOK, _DIM), jnp.float32),
        scratch_types=[
            pltpu.VMEM((_NCH, _CHUNK), jnp.int32),
            pltpu.VMEM((_NCH, _CHUNK, _DIM), jnp.float32),
            pltpu.SemaphoreType.DMA,
        ],
    )
    def _sc_gather(table_hbm, idx_hbm, out_hbm, idx_v, rows_v, sem):
        wid = lax.axis_index("s") * _SC_NC + lax.axis_index("c")
        base = wid * _B_PER_W
        for j in range(_NCH):
            pltpu.sync_copy(idx_hbm.at[pl.ds(base + j * _CHUNK, _CHUNK)], idx_v.at[j])
        copies = [
            pltpu.async_copy(table_hbm.at[idx_v.at[j]], rows_v.at[j], sem)
            for j in range(_NCH)
        ]
        for c in copies:
            c.wait()
        for j in range(_NCH):
            pltpu.sync_copy(rows_v.at[j], out_hbm.at[pl.ds(base + j * _CHUNK, _CHUNK)])

    return _sc_gather(emb, ids)


def _st_loss_body(ze_ref, zq_ref, out_ref, loss_ref, acc_ref):
    n = pl.program_id(0)
    ze = ze_ref[...]
    zq = zq_ref[...]
    out_ref[...] = (zq - ze) + ze
    part = jnp.sum((ze - zq) ** 2)
    acc_ref[0] = jnp.where(n == 0, part, acc_ref[0] + part)
    m = acc_ref[0] / jnp.float32(_N_TOK * _DIM)
    loss_ref[...] = jnp.broadcast_to(m + _BETA * m, (1, 1))


def _st_loss(ze, zq_rows):
    num_n = _N_TOK // _BNC
    return pl.pallas_call(
        _st_loss_body,
        grid=(num_n,),
        in_specs=[
            pl.BlockSpec((_BNC, _DIM), lambda n: (n, 0)),
            pl.BlockSpec((_BNC, _DIM), lambda n: (n, 0)),
        ],
        out_specs=[
            pl.BlockSpec((_BNC, _DIM), lambda n: (n, 0)),
            pl.BlockSpec((1, 1), lambda n: (0, 0)),
        ],
        out_shape=[
            jax.ShapeDtypeStruct((_N_TOK, _DIM), jnp.float32),
            jax.ShapeDtypeStruct((1, 1), jnp.float32),
        ],
        scratch_shapes=[pltpu.SMEM((1,), jnp.float32)],
    )(ze, zq_rows)


def kernel(x, emb_weight):
    b, c, h, w = x.shape
    ze = jnp.transpose(x, (0, 2, 3, 1)).reshape(-1, c)
    a = jnp.sum(ze ** 2, axis=-1, keepdims=True)
    bnorm = jnp.sum(emb_weight ** 2, axis=-1).reshape(1, -1)
    colf = jnp.arange(_K_CAT, dtype=jnp.float32).reshape(1, -1)
    ids = _compute_ids(a, bnorm, colf, ze, emb_weight).reshape(-1)
    zq_rows = _gather_rows(emb_weight, ids)
    zq_out_rows, loss = _st_loss(ze, zq_rows)
    zq_out = zq_out_rows.reshape(b, h, w, c).transpose(0, 3, 1, 2)
    return (zq_out, loss.reshape(()))
